# baseline (speedup 1.0000x reference)
import functools

import jax
import jax.numpy as jnp
from jax import lax
from jax.experimental import pallas as pl
from jax.experimental.pallas import tpu as pltpu
from jax.experimental.pallas import tpu_sc as plsc

B = 4096
L = 50
LP = 56
D = 64
NR = 5
H = 2

BB = 256
NB = B // BB


NW = 32
RPW = (B * LP) // NW
CH = 512
NCH = RPW // CH
UPW = B // NW


def _sc_gather(i2e, hist_flat, u2e, nodes):
    mesh = plsc.VectorSubcoreMesh(core_axis_name="c", subcore_axis_name="s")

    @functools.partial(
        pl.kernel,
        mesh=mesh,
        compiler_params=pltpu.CompilerParams(use_tc_tiling_on_sc=False),
        out_type=(
            jax.ShapeDtypeStruct((B * LP, D), jnp.float32),
            jax.ShapeDtypeStruct((B, D), jnp.float32),
        ),
        scratch_types=[
            pltpu.VMEM((CH,), jnp.int32),
            pltpu.VMEM((CH, D), jnp.float32),
            pltpu.VMEM((CH,), jnp.int32),
            pltpu.VMEM((CH, D), jnp.float32),
            pltpu.VMEM((UPW,), jnp.int32),
            pltpu.VMEM((UPW, D), jnp.float32),
            pltpu.SemaphoreType.DMA,
            pltpu.SemaphoreType.DMA,
        ],
    )
    def k(i2e_h, hist_h, u2e_h, nodes_h, eui_h, urep_h,
          idx0, rows0, idx1, rows1, nidx, nrows, sem0, sem1):
        wid = lax.axis_index("s") * 2 + lax.axis_index("c")
        base = wid * RPW
        idx_v = (idx0, idx1)
        rows_v = (rows0, rows1)
        sems = (sem0, sem1)

        pltpu.sync_copy(hist_h.at[pl.ds(base, CH)], idx0)
        g_prev = pltpu.async_copy(i2e_h.at[idx0], rows0, sem0)
        for c in range(NCH):
            nxt = (c + 1) % 2
            cur = c % 2
            if c + 1 < NCH:
                off = base + (c + 1) * CH
                pltpu.sync_copy(hist_h.at[pl.ds(off, CH)], idx_v[nxt])
                g_next = pltpu.async_copy(i2e_h.at[idx_v[nxt]], rows_v[nxt],
                                          sems[nxt])
            g_prev.wait()
            pltpu.sync_copy(rows_v[cur], eui_h.at[pl.ds(base + c * CH, CH)])
            if c + 1 < NCH:
                g_prev = g_next

        ub = wid * UPW
        pltpu.sync_copy(nodes_h.at[pl.ds(ub, UPW)], nidx)
        pltpu.async_copy(u2e_h.at[nidx], nrows, sem0).wait()
        pltpu.sync_copy(nrows, urep_h.at[pl.ds(ub, UPW)])

    return k(i2e, hist_flat, u2e, nodes)


_SELU_L = 1.0507009873554805
_SELU_A = 1.6732632423543772


def _selu(x):
    return _SELU_L * jnp.where(x > 0, x, _SELU_A * (jnp.exp(x) - 1.0))


def _attention_body(eui_ref, urep_ref, rat_ref, r2e_ref, gW_ref, gb_ref,
                    l1W_ref, l1b_ref, a1W_ref, a1b_ref, a2W_ref, a2b_ref,
                    a3W_ref, a3b_ref, out_ref):
    f32 = jnp.float32
    bc3 = lambda v, shape, dims: lax.broadcast_in_dim(v, shape, dims)
    e_ui = eui_ref[...]
    rat3 = rat_ref[...]
    e_r3 = jnp.zeros((BB, LP, D), f32)
    for r in range(NR):
        m = rat3 == r
        e_r3 = e_r3 + jnp.where(m, 1.0, 0.0) * bc3(r2e_ref[r], (BB, LP, D),
                                                   (2,))
    e_r = e_r3.reshape(BB * LP, D)
    prod = e_ui * e_r
    xg = (e_ui @ gW_ref[0:D, :] + e_r @ gW_ref[D:2 * D, :]
          + prod @ gW_ref[2 * D:3 * D, :] + gb_ref[...])
    alpha = jax.nn.sigmoid(xg)
    o = alpha * e_ui + (1.0 - alpha) * e_r
    ui = urep_ref[...]

    valid = lax.broadcasted_iota(jnp.int32, (BB, LP), 1) < L

    feats = []
    for h in range(H):
        o = o / jnp.maximum(
            jnp.sqrt(jnp.sum(o * o, axis=-1, keepdims=True)), 1e-12)
        ui = ui / jnp.maximum(
            jnp.sqrt(jnp.sum(ui * ui, axis=-1, keepdims=True)), 1e-12)
        o3 = o.reshape(BB, LP, D)
        a1 = a1W_ref[h]
        t = (o @ a1[0:D, :]).reshape(BB, LP, D)
        tu = ui @ a1[D:2 * D, :]
        t = _selu(t + bc3(tu, (BB, LP, D), (0, 2))
                  + bc3(a1b_ref[h], (BB, LP, D), (2,)))
        t = _selu(t.reshape(BB * LP, D) @ a2W_ref[h] + a2b_ref[h:h + 1, :])
        t3 = t.reshape(BB, LP, D // 4)
        sc = (jnp.sum(t3 * bc3(a3W_ref[h], (BB, LP, D // 4), (2,)), axis=-1)
              + a3b_ref[h, 0])
        w = (jax.nn.sigmoid(
            jnp.sum(o3 * bc3(l1W_ref[h], (BB, LP, D), (2,)), axis=-1)
            + l1b_ref[h, 0]) + 1.0)

        sc = jnp.where(valid, sc, -1e30)
        am1 = w - 1.0
        z = am1 * sc
        ex = 1.0 / am1

        def p_of(tau):
            u = z - tau
            m = u > 0
            us = jnp.where(m, u, 1.0)
            return jnp.where(m, jnp.exp(ex * jnp.log(us)), 0.0)

        tau_hi = jnp.max(z, axis=-1, keepdims=True)
        tau_lo = tau_hi - 1.0
        for _ in range(30):
            tau_m = 0.5 * (tau_lo + tau_hi)
            f = jnp.sum(p_of(tau_m), axis=-1, keepdims=True) - 1.0
            gt = f > 0
            tau_lo = jnp.where(gt, tau_m, tau_lo)
            tau_hi = jnp.where(gt, tau_hi, tau_m)
        p = p_of(0.5 * (tau_lo + tau_hi))
        att = p / jnp.maximum(jnp.sum(p, axis=-1, keepdims=True), 1e-12)

        att3 = lax.broadcast_in_dim(att, (BB, LP, D), (0, 1))
        ui = jnp.sum(att3 * o3, axis=1)
        feats.append(ui)

    out_ref[...] = (feats[0] + feats[1]) * (1.0 / H)


def _attention(eui_flat, urep, ratings_p, r2e, gate_W, gate_b,
               lin1_W, lin1_b, att1_W, att1_b, att2_W, att2_b,
               att3_W, att3_b):
    full = lambda *shape: pl.BlockSpec(shape, lambda i: (0,) * len(shape))
    return pl.pallas_call(
        _attention_body,
        grid=(NB,),
        in_specs=[
            pl.BlockSpec((BB * LP, D), lambda i: (i, 0)),
            pl.BlockSpec((BB, D), lambda i: (i, 0)),
            pl.BlockSpec((BB, LP, 1), lambda i: (i, 0, 0)),
            full(NR, D),
            full(3 * D, D),
            full(1, D),
            full(H, D),
            full(H, 1),
            full(H, 2 * D, D),
            full(H, D),
            full(H, D, D // 4),
            full(H, D // 4),
            full(H, D // 4),
            full(H, 1),
        ],
        out_specs=pl.BlockSpec((BB, D), lambda i: (i, 0)),
        out_shape=jax.ShapeDtypeStruct((B, D), jnp.float32),
    )(eui_flat, urep, ratings_p, r2e, gate_W, gate_b, lin1_W, lin1_b,
      att1_W, att1_b, att2_W, att2_b, att3_W, att3_b)


def _mlp_body(ah_ref, sf_ref, bng_ref, bnb_ref, ipW_ref, ipb_ref,
              bn1g_ref, bn1b_ref, opW_ref, opb_ref, g1W_ref, g1b_ref,
              out_ref):
    def bn(x, g, b):
        mu = jnp.mean(x, axis=0, keepdims=True)
        var = jnp.mean((x - mu) ** 2, axis=0, keepdims=True)
        return (x - mu) / jnp.sqrt(var + 1e-5) * g + b

    nf = bn(ah_ref[...], bng_ref[...], bnb_ref[...])
    nf = _selu(nf @ ipW_ref[...] + ipb_ref[...])
    nf = bn(nf, bn1g_ref[...], bn1b_ref[...])
    nf = nf @ opW_ref[...] + opb_ref[...]
    sf = sf_ref[...]
    beta = jax.nn.sigmoid(
        sf @ g1W_ref[0:D, :] + nf @ g1W_ref[D:2 * D, :]
        + (sf * nf) @ g1W_ref[2 * D:3 * D, :] + g1b_ref[...])
    out_ref[...] = beta * sf + (1.0 - beta) * nf


def _mlp_tail(att_hist, self_f, bn_g, bn_b, ip_W, ip_b, bn1_g, bn1_b,
              op_W, op_b, gate1_W, gate1_b):
    return pl.pallas_call(
        _mlp_body,
        out_shape=jax.ShapeDtypeStruct((B, D), jnp.float32),
    )(att_hist, self_f, bn_g, bn_b, ip_W, ip_b, bn1_g, bn1_b,
      op_W, op_b, gate1_W, gate1_b)


def kernel(nodes, history, ratings, u2e, i2e, r2e, gate_W, gate_b,
           lin1_W, lin1_b, att1_W, att1_b, att2_W, att2_b, att3_W, att3_b,
           bn_g, bn_b, ip_W, ip_b, bn1_g, bn1_b, op_W, op_b,
           gate1_W, gate1_b):
    f32 = jnp.float32
    i32 = jnp.int32
    nodes = nodes.astype(i32)
    history = history.astype(i32)
    ratings = ratings.astype(i32)
    u2e = u2e.astype(f32)
    i2e = i2e.astype(f32)
    r2e = r2e.astype(f32)

    hist_p = jnp.pad(history, ((0, 0), (0, LP - L))).reshape(B * LP)
    rat_p = jnp.pad(ratings, ((0, 0), (0, LP - L))).reshape(B, LP, 1)

    eui_flat, urep = _sc_gather(i2e, hist_p, u2e, nodes)

    att_hist = _attention(
        eui_flat, urep, rat_p, r2e,
        gate_W.astype(f32), gate_b.astype(f32).reshape(1, D),
        lin1_W.astype(f32).reshape(H, D), lin1_b.astype(f32).reshape(H, 1),
        att1_W.astype(f32), att1_b.astype(f32),
        att2_W.astype(f32), att2_b.astype(f32),
        att3_W.astype(f32).reshape(H, D // 4),
        att3_b.astype(f32).reshape(H, 1))

    return _mlp_tail(
        att_hist, urep,
        bn_g.astype(f32).reshape(1, D), bn_b.astype(f32).reshape(1, D),
        ip_W.astype(f32), ip_b.astype(f32).reshape(1, D),
        bn1_g.astype(f32).reshape(1, D), bn1_b.astype(f32).reshape(1, D),
        op_W.astype(f32), op_b.astype(f32).reshape(1, D),
        gate1_W.astype(f32), gate1_b.astype(f32).reshape(1, D))


# speedup vs baseline: 2.8036x; 2.8036x over previous
import functools

import jax
import jax.numpy as jnp
from jax import lax
from jax.experimental import pallas as pl
from jax.experimental.pallas import tpu as pltpu
from jax.experimental.pallas import tpu_sc as plsc

B = 4096
L = 50
LP = 56
D = 64
NR = 5
H = 2

BB = 256
NB = B // BB


NW = 32
RPW = (B * LP) // NW
CH = 512
NCH = RPW // CH
UPW = B // NW


def _sc_gather(i2e, hist_flat, u2e, nodes):
    mesh = plsc.VectorSubcoreMesh(core_axis_name="c", subcore_axis_name="s")

    @functools.partial(
        pl.kernel,
        mesh=mesh,
        compiler_params=pltpu.CompilerParams(use_tc_tiling_on_sc=False),
        out_type=(
            jax.ShapeDtypeStruct((B * LP, D), jnp.float32),
            jax.ShapeDtypeStruct((B, D), jnp.float32),
        ),
        scratch_types=[
            pltpu.VMEM((CH,), jnp.int32),
            pltpu.VMEM((CH, D), jnp.float32),
            pltpu.VMEM((CH,), jnp.int32),
            pltpu.VMEM((CH, D), jnp.float32),
            pltpu.VMEM((UPW,), jnp.int32),
            pltpu.VMEM((UPW, D), jnp.float32),
            pltpu.SemaphoreType.DMA,
            pltpu.SemaphoreType.DMA,
        ],
    )
    def k(i2e_h, hist_h, u2e_h, nodes_h, eui_h, urep_h,
          idx0, rows0, idx1, rows1, nidx, nrows, sem0, sem1):
        wid = lax.axis_index("s") * 2 + lax.axis_index("c")
        base = wid * RPW
        idx_v = (idx0, idx1)
        rows_v = (rows0, rows1)
        sems = (sem0, sem1)

        pltpu.sync_copy(hist_h.at[pl.ds(base, CH)], idx0)
        g_prev = pltpu.async_copy(i2e_h.at[idx0], rows0, sem0)
        for c in range(NCH):
            nxt = (c + 1) % 2
            cur = c % 2
            if c + 1 < NCH:
                off = base + (c + 1) * CH
                pltpu.sync_copy(hist_h.at[pl.ds(off, CH)], idx_v[nxt])
                g_next = pltpu.async_copy(i2e_h.at[idx_v[nxt]], rows_v[nxt],
                                          sems[nxt])
            g_prev.wait()
            pltpu.sync_copy(rows_v[cur], eui_h.at[pl.ds(base + c * CH, CH)])
            if c + 1 < NCH:
                g_prev = g_next

        ub = wid * UPW
        pltpu.sync_copy(nodes_h.at[pl.ds(ub, UPW)], nidx)
        pltpu.async_copy(u2e_h.at[nidx], nrows, sem0).wait()
        pltpu.sync_copy(nrows, urep_h.at[pl.ds(ub, UPW)])

    return k(i2e, hist_flat, u2e, nodes)


_SELU_L = 1.0507009873554805
_SELU_A = 1.6732632423543772


def _selu(x):
    return _SELU_L * jnp.where(x > 0, x, _SELU_A * (jnp.exp(x) - 1.0))


def _attention_body(eui_ref, urep_ref, rat_ref, r2e_ref, gW_ref, gb_ref,
                    l1W_ref, l1b_ref, a1W_ref, a1b_ref, a2W_ref, a2b_ref,
                    a3W_ref, a3b_ref, out_ref):
    f32 = jnp.float32
    bc3 = lambda v, shape, dims: lax.broadcast_in_dim(v, shape, dims)
    e_ui = eui_ref[...]
    rat3 = rat_ref[...]
    e_r3 = jnp.zeros((BB, LP, D), f32)
    for r in range(NR):
        m = rat3 == r
        e_r3 = e_r3 + jnp.where(m, 1.0, 0.0) * bc3(r2e_ref[r], (BB, LP, D),
                                                   (2,))
    e_r = e_r3.reshape(BB * LP, D)
    prod = e_ui * e_r
    xg = (e_ui @ gW_ref[0:D, :] + e_r @ gW_ref[D:2 * D, :]
          + prod @ gW_ref[2 * D:3 * D, :] + gb_ref[...])
    alpha = jax.nn.sigmoid(xg)
    o = alpha * e_ui + (1.0 - alpha) * e_r
    ui = urep_ref[...]

    validT = lax.broadcasted_iota(jnp.int32, (LP, BB), 0) < L

    feats = []
    for h in range(H):
        o = o / jnp.maximum(
            jnp.sqrt(jnp.sum(o * o, axis=-1, keepdims=True)), 1e-12)
        ui = ui / jnp.maximum(
            jnp.sqrt(jnp.sum(ui * ui, axis=-1, keepdims=True)), 1e-12)
        o3 = o.reshape(BB, LP, D)
        a1 = a1W_ref[h]
        t = (o @ a1[0:D, :]).reshape(BB, LP, D)
        tu = ui @ a1[D:2 * D, :]
        t = _selu(t + bc3(tu, (BB, LP, D), (0, 2))
                  + bc3(a1b_ref[h], (BB, LP, D), (2,)))
        t = _selu(t.reshape(BB * LP, D) @ a2W_ref[h] + a2b_ref[h:h + 1, :])
        t3 = t.reshape(BB, LP, D // 4)
        sc = (jnp.sum(t3 * bc3(a3W_ref[h], (BB, LP, D // 4), (2,)), axis=-1)
              + a3b_ref[h, 0])
        w = (jax.nn.sigmoid(
            jnp.sum(o3 * bc3(l1W_ref[h], (BB, LP, D), (2,)), axis=-1)
            + l1b_ref[h, 0]) + 1.0)

        scT = jnp.where(validT, sc.T, -1e30)
        am1 = w.T - 1.0
        z = am1 * scT
        ex = 1.0 / am1

        def p_of(tau):
            u = z - tau
            m = u > 0
            us = jnp.where(m, u, 1.0)
            return jnp.where(m, jnp.exp(ex * jnp.log(us)), 0.0)

        tau_hi = jnp.max(z, axis=0, keepdims=True)
        tau_lo = tau_hi - 1.0
        for _ in range(16):
            tau_m = 0.5 * (tau_lo + tau_hi)
            f = jnp.sum(p_of(tau_m), axis=0, keepdims=True) - 1.0
            gt = f > 0
            tau_lo = jnp.where(gt, tau_m, tau_lo)
            tau_hi = jnp.where(gt, tau_hi, tau_m)
        p = p_of(0.5 * (tau_lo + tau_hi))
        att = (p / jnp.maximum(jnp.sum(p, axis=0, keepdims=True), 1e-12)).T

        att3 = lax.broadcast_in_dim(att, (BB, LP, D), (0, 1))
        ui = jnp.sum(att3 * o3, axis=1)
        feats.append(ui)

    out_ref[...] = (feats[0] + feats[1]) * (1.0 / H)


def _attention(eui_flat, urep, ratings_p, r2e, gate_W, gate_b,
               lin1_W, lin1_b, att1_W, att1_b, att2_W, att2_b,
               att3_W, att3_b):
    full = lambda *shape: pl.BlockSpec(shape, lambda i: (0,) * len(shape))
    return pl.pallas_call(
        _attention_body,
        grid=(NB,),
        in_specs=[
            pl.BlockSpec((BB * LP, D), lambda i: (i, 0)),
            pl.BlockSpec((BB, D), lambda i: (i, 0)),
            pl.BlockSpec((BB, LP, 1), lambda i: (i, 0, 0)),
            full(NR, D),
            full(3 * D, D),
            full(1, D),
            full(H, D),
            full(H, 1),
            full(H, 2 * D, D),
            full(H, D),
            full(H, D, D // 4),
            full(H, D // 4),
            full(H, D // 4),
            full(H, 1),
        ],
        out_specs=pl.BlockSpec((BB, D), lambda i: (i, 0)),
        out_shape=jax.ShapeDtypeStruct((B, D), jnp.float32),
    )(eui_flat, urep, ratings_p, r2e, gate_W, gate_b, lin1_W, lin1_b,
      att1_W, att1_b, att2_W, att2_b, att3_W, att3_b)


def _mlp_body(ah_ref, sf_ref, bng_ref, bnb_ref, ipW_ref, ipb_ref,
              bn1g_ref, bn1b_ref, opW_ref, opb_ref, g1W_ref, g1b_ref,
              out_ref):
    def bn(x, g, b):
        mu = jnp.mean(x, axis=0, keepdims=True)
        var = jnp.mean((x - mu) ** 2, axis=0, keepdims=True)
        return (x - mu) / jnp.sqrt(var + 1e-5) * g + b

    nf = bn(ah_ref[...], bng_ref[...], bnb_ref[...])
    nf = _selu(nf @ ipW_ref[...] + ipb_ref[...])
    nf = bn(nf, bn1g_ref[...], bn1b_ref[...])
    nf = nf @ opW_ref[...] + opb_ref[...]
    sf = sf_ref[...]
    beta = jax.nn.sigmoid(
        sf @ g1W_ref[0:D, :] + nf @ g1W_ref[D:2 * D, :]
        + (sf * nf) @ g1W_ref[2 * D:3 * D, :] + g1b_ref[...])
    out_ref[...] = beta * sf + (1.0 - beta) * nf


def _mlp_tail(att_hist, self_f, bn_g, bn_b, ip_W, ip_b, bn1_g, bn1_b,
              op_W, op_b, gate1_W, gate1_b):
    return pl.pallas_call(
        _mlp_body,
        out_shape=jax.ShapeDtypeStruct((B, D), jnp.float32),
    )(att_hist, self_f, bn_g, bn_b, ip_W, ip_b, bn1_g, bn1_b,
      op_W, op_b, gate1_W, gate1_b)


def kernel(nodes, history, ratings, u2e, i2e, r2e, gate_W, gate_b,
           lin1_W, lin1_b, att1_W, att1_b, att2_W, att2_b, att3_W, att3_b,
           bn_g, bn_b, ip_W, ip_b, bn1_g, bn1_b, op_W, op_b,
           gate1_W, gate1_b):
    f32 = jnp.float32
    i32 = jnp.int32
    nodes = nodes.astype(i32)
    history = history.astype(i32)
    ratings = ratings.astype(i32)
    u2e = u2e.astype(f32)
    i2e = i2e.astype(f32)
    r2e = r2e.astype(f32)

    hist_p = jnp.pad(history, ((0, 0), (0, LP - L))).reshape(B * LP)
    rat_p = jnp.pad(ratings, ((0, 0), (0, LP - L))).reshape(B, LP, 1)

    eui_flat, urep = _sc_gather(i2e, hist_p, u2e, nodes)

    att_hist = _attention(
        eui_flat, urep, rat_p, r2e,
        gate_W.astype(f32), gate_b.astype(f32).reshape(1, D),
        lin1_W.astype(f32).reshape(H, D), lin1_b.astype(f32).reshape(H, 1),
        att1_W.astype(f32), att1_b.astype(f32),
        att2_W.astype(f32), att2_b.astype(f32),
        att3_W.astype(f32).reshape(H, D // 4),
        att3_b.astype(f32).reshape(H, 1))

    return _mlp_tail(
        att_hist, urep,
        bn_g.astype(f32).reshape(1, D), bn_b.astype(f32).reshape(1, D),
        ip_W.astype(f32), ip_b.astype(f32).reshape(1, D),
        bn1_g.astype(f32).reshape(1, D), bn1_b.astype(f32).reshape(1, D),
        op_W.astype(f32), op_b.astype(f32).reshape(1, D),
        gate1_W.astype(f32), gate1_b.astype(f32).reshape(1, D))


# speedup vs baseline: 2.8497x; 1.0165x over previous
import functools

import jax
import jax.numpy as jnp
from jax import lax
from jax.experimental import pallas as pl
from jax.experimental.pallas import tpu as pltpu
from jax.experimental.pallas import tpu_sc as plsc

B = 4096
L = 50
LP = 56
D = 64
NR = 5
H = 2

BB = 256
NB = B // BB


NW = 32
RPW = (B * LP) // NW
CH = 512
NCH = RPW // CH
UPW = B // NW


def _sc_gather(i2e, hist_flat, u2e, nodes):
    mesh = plsc.VectorSubcoreMesh(core_axis_name="c", subcore_axis_name="s")

    @functools.partial(
        pl.kernel,
        mesh=mesh,
        compiler_params=pltpu.CompilerParams(use_tc_tiling_on_sc=False),
        out_type=(
            jax.ShapeDtypeStruct((B * LP, D), jnp.float32),
            jax.ShapeDtypeStruct((B, D), jnp.float32),
        ),
        scratch_types=[
            pltpu.VMEM((CH,), jnp.int32),
            pltpu.VMEM((CH, D), jnp.float32),
            pltpu.VMEM((CH,), jnp.int32),
            pltpu.VMEM((CH, D), jnp.float32),
            pltpu.VMEM((UPW,), jnp.int32),
            pltpu.VMEM((UPW, D), jnp.float32),
            pltpu.SemaphoreType.DMA,
            pltpu.SemaphoreType.DMA,
        ],
    )
    def k(i2e_h, hist_h, u2e_h, nodes_h, eui_h, urep_h,
          idx0, rows0, idx1, rows1, nidx, nrows, sem0, sem1):
        wid = lax.axis_index("s") * 2 + lax.axis_index("c")
        base = wid * RPW
        idx_v = (idx0, idx1)
        rows_v = (rows0, rows1)
        sems = (sem0, sem1)

        pltpu.sync_copy(hist_h.at[pl.ds(base, CH)], idx0)
        g_prev = pltpu.async_copy(i2e_h.at[idx0], rows0, sem0)
        for c in range(NCH):
            nxt = (c + 1) % 2
            cur = c % 2
            if c + 1 < NCH:
                off = base + (c + 1) * CH
                pltpu.sync_copy(hist_h.at[pl.ds(off, CH)], idx_v[nxt])
                g_next = pltpu.async_copy(i2e_h.at[idx_v[nxt]], rows_v[nxt],
                                          sems[nxt])
            g_prev.wait()
            pltpu.sync_copy(rows_v[cur], eui_h.at[pl.ds(base + c * CH, CH)])
            if c + 1 < NCH:
                g_prev = g_next

        ub = wid * UPW
        pltpu.sync_copy(nodes_h.at[pl.ds(ub, UPW)], nidx)
        pltpu.async_copy(u2e_h.at[nidx], nrows, sem0).wait()
        pltpu.sync_copy(nrows, urep_h.at[pl.ds(ub, UPW)])

    return k(i2e, hist_flat, u2e, nodes)


_SELU_L = 1.0507009873554805
_SELU_A = 1.6732632423543772


def _selu(x):
    return _SELU_L * jnp.where(x > 0, x, _SELU_A * (jnp.exp(x) - 1.0))


def _attention_body(eui_ref, urep_ref, rat_ref, r2e_ref, gW_ref, gb_ref,
                    l1W_ref, l1b_ref, a1W_ref, a1b_ref, a2W_ref, a2b_ref,
                    a3W_ref, a3b_ref, out_ref):
    bc3 = lambda v, shape, dims: lax.broadcast_in_dim(v, shape, dims)
    e_ui = eui_ref[...]
    e_r = rat_ref[...] @ r2e_ref[...]
    prod = e_ui * e_r
    xg = (e_ui @ gW_ref[0:D, :] + e_r @ gW_ref[D:2 * D, :]
          + prod @ gW_ref[2 * D:3 * D, :] + gb_ref[...])
    alpha = jax.nn.sigmoid(xg)
    o = alpha * e_ui + (1.0 - alpha) * e_r
    ui = urep_ref[...]

    validT = lax.broadcasted_iota(jnp.int32, (LP, BB), 0) < L

    feats = []
    for h in range(H):
        o = o / jnp.maximum(
            jnp.sqrt(jnp.sum(o * o, axis=-1, keepdims=True)), 1e-12)
        ui = ui / jnp.maximum(
            jnp.sqrt(jnp.sum(ui * ui, axis=-1, keepdims=True)), 1e-12)
        o3 = o.reshape(BB, LP, D)
        a1 = a1W_ref[h]
        t = (o @ a1[0:D, :]).reshape(BB, LP, D)
        tu = ui @ a1[D:2 * D, :]
        t = _selu(t + bc3(tu, (BB, LP, D), (0, 2))
                  + bc3(a1b_ref[h], (BB, LP, D), (2,)))
        t = _selu(t.reshape(BB * LP, D) @ a2W_ref[h] + a2b_ref[h:h + 1, :])
        t3 = t.reshape(BB, LP, D // 4)
        sc = (jnp.sum(t3 * bc3(a3W_ref[h], (BB, LP, D // 4), (2,)), axis=-1)
              + a3b_ref[h, 0])
        w = (jax.nn.sigmoid(
            jnp.sum(o3 * bc3(l1W_ref[h], (BB, LP, D), (2,)), axis=-1)
            + l1b_ref[h, 0]) + 1.0)

        scT = jnp.where(validT, sc.T, -1e30)
        am1 = w.T - 1.0
        z = am1 * scT
        ex = 1.0 / am1

        def p_of(tau):
            u = z - tau
            m = u > 0
            us = jnp.where(m, u, 1.0)
            return jnp.where(m, jnp.exp(ex * jnp.log(us)), 0.0)

        tau_hi = jnp.max(z, axis=0, keepdims=True)
        tau_lo = tau_hi - 1.0
        for _ in range(16):
            tau_m = 0.5 * (tau_lo + tau_hi)
            f = jnp.sum(p_of(tau_m), axis=0, keepdims=True) - 1.0
            gt = f > 0
            tau_lo = jnp.where(gt, tau_m, tau_lo)
            tau_hi = jnp.where(gt, tau_hi, tau_m)
        p = p_of(0.5 * (tau_lo + tau_hi))
        att = (p / jnp.maximum(jnp.sum(p, axis=0, keepdims=True), 1e-12)).T

        att3 = lax.broadcast_in_dim(att, (BB, LP, D), (0, 1))
        ui = jnp.sum(att3 * o3, axis=1)
        feats.append(ui)

    out_ref[...] = (feats[0] + feats[1]) * (1.0 / H)


def _attention(eui_flat, urep, ratings_p, r2e, gate_W, gate_b,
               lin1_W, lin1_b, att1_W, att1_b, att2_W, att2_b,
               att3_W, att3_b):
    full = lambda *shape: pl.BlockSpec(shape, lambda i: (0,) * len(shape))
    return pl.pallas_call(
        _attention_body,
        grid=(NB,),
        in_specs=[
            pl.BlockSpec((BB * LP, D), lambda i: (i, 0)),
            pl.BlockSpec((BB, D), lambda i: (i, 0)),
            pl.BlockSpec((BB * LP, 8), lambda i: (i, 0)),
            full(8, D),
            full(3 * D, D),
            full(1, D),
            full(H, D),
            full(H, 1),
            full(H, 2 * D, D),
            full(H, D),
            full(H, D, D // 4),
            full(H, D // 4),
            full(H, D // 4),
            full(H, 1),
        ],
        out_specs=pl.BlockSpec((BB, D), lambda i: (i, 0)),
        out_shape=jax.ShapeDtypeStruct((B, D), jnp.float32),
    )(eui_flat, urep, ratings_p, r2e, gate_W, gate_b, lin1_W, lin1_b,
      att1_W, att1_b, att2_W, att2_b, att3_W, att3_b)


def _mlp_body(ah_ref, sf_ref, bng_ref, bnb_ref, ipW_ref, ipb_ref,
              bn1g_ref, bn1b_ref, opW_ref, opb_ref, g1W_ref, g1b_ref,
              out_ref):
    def bn(x, g, b):
        mu = jnp.mean(x, axis=0, keepdims=True)
        var = jnp.mean((x - mu) ** 2, axis=0, keepdims=True)
        return (x - mu) / jnp.sqrt(var + 1e-5) * g + b

    nf = bn(ah_ref[...], bng_ref[...], bnb_ref[...])
    nf = _selu(nf @ ipW_ref[...] + ipb_ref[...])
    nf = bn(nf, bn1g_ref[...], bn1b_ref[...])
    nf = nf @ opW_ref[...] + opb_ref[...]
    sf = sf_ref[...]
    beta = jax.nn.sigmoid(
        sf @ g1W_ref[0:D, :] + nf @ g1W_ref[D:2 * D, :]
        + (sf * nf) @ g1W_ref[2 * D:3 * D, :] + g1b_ref[...])
    out_ref[...] = beta * sf + (1.0 - beta) * nf


def _mlp_tail(att_hist, self_f, bn_g, bn_b, ip_W, ip_b, bn1_g, bn1_b,
              op_W, op_b, gate1_W, gate1_b):
    return pl.pallas_call(
        _mlp_body,
        out_shape=jax.ShapeDtypeStruct((B, D), jnp.float32),
    )(att_hist, self_f, bn_g, bn_b, ip_W, ip_b, bn1_g, bn1_b,
      op_W, op_b, gate1_W, gate1_b)


def kernel(nodes, history, ratings, u2e, i2e, r2e, gate_W, gate_b,
           lin1_W, lin1_b, att1_W, att1_b, att2_W, att2_b, att3_W, att3_b,
           bn_g, bn_b, ip_W, ip_b, bn1_g, bn1_b, op_W, op_b,
           gate1_W, gate1_b):
    f32 = jnp.float32
    i32 = jnp.int32
    nodes = nodes.astype(i32)
    history = history.astype(i32)
    ratings = ratings.astype(i32)
    u2e = u2e.astype(f32)
    i2e = i2e.astype(f32)
    r2e = r2e.astype(f32)

    hist_p = jnp.pad(history, ((0, 0), (0, LP - L))).reshape(B * LP)
    oh = (ratings[..., None] == jnp.arange(NR, dtype=i32)).astype(f32)
    rat_p = jnp.pad(oh, ((0, 0), (0, LP - L), (0, 8 - NR))).reshape(B * LP, 8)
    r2e = jnp.pad(r2e, ((0, 8 - NR), (0, 0)))

    eui_flat, urep = _sc_gather(i2e, hist_p, u2e, nodes)

    att_hist = _attention(
        eui_flat, urep, rat_p, r2e,
        gate_W.astype(f32), gate_b.astype(f32).reshape(1, D),
        lin1_W.astype(f32).reshape(H, D), lin1_b.astype(f32).reshape(H, 1),
        att1_W.astype(f32), att1_b.astype(f32),
        att2_W.astype(f32), att2_b.astype(f32),
        att3_W.astype(f32).reshape(H, D // 4),
        att3_b.astype(f32).reshape(H, 1))

    return _mlp_tail(
        att_hist, urep,
        bn_g.astype(f32).reshape(1, D), bn_b.astype(f32).reshape(1, D),
        ip_W.astype(f32), ip_b.astype(f32).reshape(1, D),
        bn1_g.astype(f32).reshape(1, D), bn1_b.astype(f32).reshape(1, D),
        op_W.astype(f32), op_b.astype(f32).reshape(1, D),
        gate1_W.astype(f32), gate1_b.astype(f32).reshape(1, D))
